# baseline (speedup 1.0000x reference)
import functools

import jax
import jax.numpy as jnp
from jax import lax
from jax.experimental import pallas as pl
from jax.experimental.pallas import tpu as pltpu
from jax.experimental.pallas import tpu_sc as plsc

_N = 10000
_E = 320000
_D = 128
_H = 128
_C = 64

_NC = 2
_NS = 16
_NW = _NC * _NS
_B = 128
_NBUF = 2
_GRP = 8
_EPW = 10240
_EPAD = _EPW * _NW
_NGG = _EPW // (_GRP * _B)
_RPS = 632
_NPAD = _RPS * _NS


def _sc_agg_body(x_hbm, src_hbm, dst2d_hbm, zeros_hbm, out_hbm,
                 src_v, dst_v, rows_v, srcw_v, roww_v, agg_sh,
                 semi0, semi1, sg0, sg1, sg2, sg3, ss0, ss1, ss2, ss3):
    del dst2d_hbm
    c = lax.axis_index("c")
    s = lax.axis_index("s")
    w = c * _NS + s
    sg = (sg0, sg1, sg2, sg3)
    ss = (ss0, ss1, ss2, ss3)

    pltpu.sync_copy(zeros_hbm.at[pl.ds(s * _RPS, _RPS)],
                    agg_sh.at[pl.ds(s * _RPS, _RPS)])
    plsc.subcore_barrier()

    def chunk(j, carry):
        ebase = w * _EPW + j * 80
        pltpu.sync_copy(src_hbm.at[pl.ds(ebase, 80)], srcw_v)
        pltpu.async_copy(x_hbm.at[srcw_v], roww_v, sg0).wait()
        return carry

    lax.fori_loop(0, 128, chunk, 0)

    plsc.subcore_barrier()
    pltpu.sync_copy(agg_sh.at[pl.ds(s * _RPS, _RPS)],
                    out_hbm.at[c, pl.ds(s * _RPS, _RPS)])


@jax.jit
def _sc_segment_sum(x, src, dst2d, zeros):
    mesh = plsc.VectorSubcoreMesh(core_axis_name="c", subcore_axis_name="s")
    f = pl.kernel(
        _sc_agg_body,
        out_type=jax.ShapeDtypeStruct((_NC, _NPAD, _D), jnp.float32),
        mesh=mesh,
        scratch_types=[
            pltpu.VMEM((_GRP * _B,), jnp.int32),
            pltpu.VMEM((_GRP, _B), jnp.int32),
            pltpu.VMEM((_NBUF, _B, _D), jnp.float32),
            pltpu.VMEM((80,), jnp.int32),
            pltpu.VMEM((80, _D), jnp.float32),
            pltpu.VMEM_SHARED((_NPAD, _D), jnp.float32),
        ] + [pltpu.SemaphoreType.DMA] * 10,
    )
    return f(x, src, dst2d, zeros)


def _mlp_body(eps_ref, x_ref, agg_ref, wa_ref, ba_ref, g_ref, be_ref,
              wb_ref, bb_ref, o_ref, *, last):
    agg = agg_ref[0, :_N, :] + agg_ref[1, :_N, :]
    h = (1.0 + eps_ref[0]) * x_ref[...] + agg
    t = jnp.dot(h, wa_ref[...], preferred_element_type=jnp.float32) + ba_ref[...]
    t = jnp.maximum(t, 0.0)
    mu = jnp.mean(t, axis=0, keepdims=True)
    var = jnp.mean((t - mu) ** 2, axis=0, keepdims=True)
    t = g_ref[...] * (t - mu) * lax.rsqrt(var + 1e-5) + be_ref[...]
    o = jnp.dot(t, wb_ref[...], preferred_element_type=jnp.float32) + bb_ref[...]
    if last:
        o = o - jnp.max(o, axis=-1, keepdims=True)
        o = o - jnp.log(jnp.sum(jnp.exp(o), axis=-1, keepdims=True))
    else:
        o = jnp.maximum(o, 0.0)
    o_ref[...] = o


def _mlp(eps, x, agg, wa, ba, g, be, wb, bb, *, last):
    cout = wb.shape[1]
    return pl.pallas_call(
        functools.partial(_mlp_body, last=last),
        out_shape=jax.ShapeDtypeStruct((_N, cout), jnp.float32),
        in_specs=[pl.BlockSpec(memory_space=pltpu.SMEM)]
        + [pl.BlockSpec(memory_space=pltpu.VMEM)] * 8,
        out_specs=pl.BlockSpec(memory_space=pltpu.VMEM),
    )(eps, x, agg, wa, ba, g, be, wb, bb)


def kernel(x, edge_index, eps1, W1a, b1a, g1, be1, W1b, b1b,
           eps2, W2a, b2a, g2, be2, W2b, b2b):
    ei = edge_index.astype(jnp.int32)
    zeros = jnp.zeros((_NPAD, _D), jnp.float32)
    e1 = jnp.reshape(eps1, (1,)).astype(jnp.float32)
    e2 = jnp.reshape(eps2, (1,)).astype(jnp.float32)

    npad_e = _EPAD - _E
    src = jnp.concatenate([ei[0], jnp.zeros((npad_e,), jnp.int32)])
    dst = jnp.concatenate(
        [ei[1], _N + (jnp.arange(npad_e, dtype=jnp.int32) % (_NPAD - _N))])
    dst2d = dst.reshape(_EPAD // _B, _B)
    agg1 = _sc_segment_sum(x, src, dst2d, zeros)
    h1 = _mlp(e1, x, agg1, W1a, b1a.reshape(1, _H), g1.reshape(1, _H),
              be1.reshape(1, _H), W1b, b1b.reshape(1, _H), last=False)
    agg2 = _sc_segment_sum(h1, src, dst2d, zeros)
    out = _mlp(e2, h1, agg2, W2a, b2a.reshape(1, _H), g2.reshape(1, _H),
               be2.reshape(1, _H), W2b, b2b.reshape(1, _C), last=True)
    return out


# speedup vs baseline: 1.6782x; 1.6782x over previous
import functools

import jax
import jax.numpy as jnp
from jax import lax
from jax.experimental import pallas as pl
from jax.experimental.pallas import tpu as pltpu
from jax.experimental.pallas import tpu_sc as plsc

_N = 10000
_E = 320000
_D = 128
_H = 128
_C = 64

_NC = 2
_NS = 16
_NW = _NC * _NS
_EPW = _E // _NW
_B = 80
_NITER = _EPW // _B
_RPS = 632
_NPAD = _RPS * _NS


def _sc_agg_body(x_hbm, src_hbm, dst_hbm, zeros_hbm, out_hbm,
                 src_v, dst_v, rows_v, agg_sh, sem):
    c = lax.axis_index("c")
    s = lax.axis_index("s")
    w = c * _NS + s

    pltpu.sync_copy(zeros_hbm.at[pl.ds(s * _RPS, _RPS)],
                    agg_sh.at[pl.ds(s * _RPS, _RPS)])
    plsc.subcore_barrier()

    def step(j, carry):
        base = w * _EPW + j * _B
        pltpu.sync_copy(src_hbm.at[pl.ds(base, _B)], src_v)
        pltpu.sync_copy(dst_hbm.at[pl.ds(base, _B)], dst_v)
        pltpu.async_copy(x_hbm.at[src_v], rows_v, sem).wait()
        pltpu.sync_copy(rows_v, agg_sh.at[dst_v], add=True)
        return carry

    lax.fori_loop(0, _NITER, step, 0)

    plsc.subcore_barrier()
    pltpu.sync_copy(agg_sh.at[pl.ds(s * _RPS, _RPS)],
                    out_hbm.at[c, pl.ds(s * _RPS, _RPS)])


@jax.jit
def _sc_segment_sum(x, src, dst, zeros):
    mesh = plsc.VectorSubcoreMesh(core_axis_name="c", subcore_axis_name="s")
    f = pl.kernel(
        _sc_agg_body,
        out_type=jax.ShapeDtypeStruct((_NC, _NPAD, _D), jnp.float32),
        mesh=mesh,
        scratch_types=[
            pltpu.VMEM((_B,), jnp.int32),
            pltpu.VMEM((_B,), jnp.int32),
            pltpu.VMEM((_B, _D), jnp.float32),
            pltpu.VMEM_SHARED((_NPAD, _D), jnp.float32),
            pltpu.SemaphoreType.DMA,
        ],
    )
    return f(x, src, dst, zeros)


def _mlp_body(eps_ref, x_ref, agg_ref, wa_ref, ba_ref, g_ref, be_ref,
              wb_ref, bb_ref, o_ref, *, last):
    agg = agg_ref[0, :_N, :] + agg_ref[1, :_N, :]
    h = (1.0 + eps_ref[0]) * x_ref[...] + agg
    t = jnp.dot(h, wa_ref[...], preferred_element_type=jnp.float32) + ba_ref[...]
    t = jnp.maximum(t, 0.0)
    mu = jnp.mean(t, axis=0, keepdims=True)
    var = jnp.mean((t - mu) ** 2, axis=0, keepdims=True)
    t = g_ref[...] * (t - mu) * lax.rsqrt(var + 1e-5) + be_ref[...]
    o = jnp.dot(t, wb_ref[...], preferred_element_type=jnp.float32) + bb_ref[...]
    if last:
        o = o - jnp.max(o, axis=-1, keepdims=True)
        o = o - jnp.log(jnp.sum(jnp.exp(o), axis=-1, keepdims=True))
    else:
        o = jnp.maximum(o, 0.0)
    o_ref[...] = o


def _mlp(eps, x, agg, wa, ba, g, be, wb, bb, *, last):
    cout = wb.shape[1]
    return pl.pallas_call(
        functools.partial(_mlp_body, last=last),
        out_shape=jax.ShapeDtypeStruct((_N, cout), jnp.float32),
        in_specs=[pl.BlockSpec(memory_space=pltpu.SMEM)]
        + [pl.BlockSpec(memory_space=pltpu.VMEM)] * 8,
        out_specs=pl.BlockSpec(memory_space=pltpu.VMEM),
    )(eps, x, agg, wa, ba, g, be, wb, bb)


def kernel(x, edge_index, eps1, W1a, b1a, g1, be1, W1b, b1b,
           eps2, W2a, b2a, g2, be2, W2b, b2b):
    ei = edge_index.astype(jnp.int32)
    zeros = jnp.zeros((_NPAD, _D), jnp.float32)
    e1 = jnp.reshape(eps1, (1,)).astype(jnp.float32)
    e2 = jnp.reshape(eps2, (1,)).astype(jnp.float32)

    src, dst = ei[0], ei[1]
    agg1 = _sc_segment_sum(x, src, dst, zeros)
    h1 = _mlp(e1, x, agg1, W1a, b1a.reshape(1, _H), g1.reshape(1, _H),
              be1.reshape(1, _H), W1b, b1b.reshape(1, _H), last=False)
    agg2 = _sc_segment_sum(h1, src, dst, zeros)
    out = _mlp(e2, h1, agg2, W2a, b2a.reshape(1, _H), g2.reshape(1, _H),
               be2.reshape(1, _H), W2b, b2b.reshape(1, _C), last=True)
    return out


# speedup vs baseline: 2.6520x; 1.5803x over previous
import functools

import jax
import jax.numpy as jnp
from jax import lax
from jax.experimental import pallas as pl
from jax.experimental.pallas import tpu as pltpu
from jax.experimental.pallas import tpu_sc as plsc

_N = 10000
_E = 320000
_D = 128
_H = 128
_C = 64

_NC = 2
_NS = 16
_NW = _NC * _NS
_EPW = _E // _NW
_B = 80
_NITER = _EPW // _B
_RPS = 632
_NPAD = _RPS * _NS


def _sc_agg_body(x_hbm, src_hbm, dst_hbm, zeros_hbm, out_hbm,
                 src0, src1, dst0, dst1, rows0, rows1, agg_sh,
                 si0, si1, sj0, sj1, sg0, sg1, ss0, ss1):
    c = lax.axis_index("c")
    s = lax.axis_index("s")
    w = c * _NS + s

    pltpu.sync_copy(zeros_hbm.at[pl.ds(s * _RPS, _RPS)],
                    agg_sh.at[pl.ds(s * _RPS, _RPS)])
    plsc.subcore_barrier()

    def pair(jj, carry):
        base0 = w * _EPW + jj * (2 * _B)
        base1 = base0 + _B
        d_s0 = pltpu.async_copy(src_hbm.at[pl.ds(base0, _B)], src0, si0)
        d_d0 = pltpu.async_copy(dst_hbm.at[pl.ds(base0, _B)], dst0, sj0)
        d_s1 = pltpu.async_copy(src_hbm.at[pl.ds(base1, _B)], src1, si1)
        d_d1 = pltpu.async_copy(dst_hbm.at[pl.ds(base1, _B)], dst1, sj1)
        d_s0.wait()
        d_g0 = pltpu.async_copy(x_hbm.at[src0], rows0, sg0)
        d_s1.wait()
        d_g1 = pltpu.async_copy(x_hbm.at[src1], rows1, sg1)
        d_g0.wait()
        d_d0.wait()
        d_c0 = pltpu.async_copy(rows0, agg_sh.at[dst0], ss0, add=True)
        d_g1.wait()
        d_d1.wait()
        d_c1 = pltpu.async_copy(rows1, agg_sh.at[dst1], ss1, add=True)
        d_c0.wait()
        d_c1.wait()
        return carry

    lax.fori_loop(0, _NITER // 2, pair, 0)

    base = w * _EPW + (_NITER - 1) * _B
    pltpu.sync_copy(src_hbm.at[pl.ds(base, _B)], src0)
    pltpu.sync_copy(dst_hbm.at[pl.ds(base, _B)], dst0)
    pltpu.async_copy(x_hbm.at[src0], rows0, sg0).wait()
    pltpu.sync_copy(rows0, agg_sh.at[dst0], add=True)

    plsc.subcore_barrier()
    pltpu.sync_copy(agg_sh.at[pl.ds(s * _RPS, _RPS)],
                    out_hbm.at[c, pl.ds(s * _RPS, _RPS)])


@jax.jit
def _sc_segment_sum(x, src, dst, zeros):
    mesh = plsc.VectorSubcoreMesh(core_axis_name="c", subcore_axis_name="s")
    f = pl.kernel(
        _sc_agg_body,
        out_type=jax.ShapeDtypeStruct((_NC, _NPAD, _D), jnp.float32),
        mesh=mesh,
        scratch_types=[
            pltpu.VMEM((_B,), jnp.int32),
            pltpu.VMEM((_B,), jnp.int32),
            pltpu.VMEM((_B,), jnp.int32),
            pltpu.VMEM((_B,), jnp.int32),
            pltpu.VMEM((_B, _D), jnp.float32),
            pltpu.VMEM((_B, _D), jnp.float32),
            pltpu.VMEM_SHARED((_NPAD, _D), jnp.float32),
        ] + [pltpu.SemaphoreType.DMA] * 8,
    )
    return f(x, src, dst, zeros)


def _mlp_body(eps_ref, x_ref, agg_ref, wa_ref, ba_ref, g_ref, be_ref,
              wb_ref, bb_ref, o_ref, *, last):
    agg = agg_ref[0, :_N, :] + agg_ref[1, :_N, :]
    h = (1.0 + eps_ref[0]) * x_ref[...] + agg
    t = jnp.dot(h, wa_ref[...], preferred_element_type=jnp.float32) + ba_ref[...]
    t = jnp.maximum(t, 0.0)
    mu = jnp.mean(t, axis=0, keepdims=True)
    var = jnp.mean((t - mu) ** 2, axis=0, keepdims=True)
    t = g_ref[...] * (t - mu) * lax.rsqrt(var + 1e-5) + be_ref[...]
    o = jnp.dot(t, wb_ref[...], preferred_element_type=jnp.float32) + bb_ref[...]
    if last:
        o = o - jnp.max(o, axis=-1, keepdims=True)
        o = o - jnp.log(jnp.sum(jnp.exp(o), axis=-1, keepdims=True))
    else:
        o = jnp.maximum(o, 0.0)
    o_ref[...] = o


def _mlp(eps, x, agg, wa, ba, g, be, wb, bb, *, last):
    cout = wb.shape[1]
    return pl.pallas_call(
        functools.partial(_mlp_body, last=last),
        out_shape=jax.ShapeDtypeStruct((_N, cout), jnp.float32),
        in_specs=[pl.BlockSpec(memory_space=pltpu.SMEM)]
        + [pl.BlockSpec(memory_space=pltpu.VMEM)] * 8,
        out_specs=pl.BlockSpec(memory_space=pltpu.VMEM),
    )(eps, x, agg, wa, ba, g, be, wb, bb)


def kernel(x, edge_index, eps1, W1a, b1a, g1, be1, W1b, b1b,
           eps2, W2a, b2a, g2, be2, W2b, b2b):
    ei = edge_index.astype(jnp.int32)
    zeros = jnp.zeros((_NPAD, _D), jnp.float32)
    e1 = jnp.reshape(eps1, (1,)).astype(jnp.float32)
    e2 = jnp.reshape(eps2, (1,)).astype(jnp.float32)

    src, dst = ei[0], ei[1]
    agg1 = _sc_segment_sum(x, src, dst, zeros)
    h1 = _mlp(e1, x, agg1, W1a, b1a.reshape(1, _H), g1.reshape(1, _H),
              be1.reshape(1, _H), W1b, b1b.reshape(1, _H), last=False)
    agg2 = _sc_segment_sum(h1, src, dst, zeros)
    out = _mlp(e2, h1, agg2, W2a, b2a.reshape(1, _H), g2.reshape(1, _H),
               be2.reshape(1, _H), W2b, b2b.reshape(1, _C), last=True)
    return out


# speedup vs baseline: 3.2543x; 1.2271x over previous
import functools

import jax
import jax.numpy as jnp
from jax import lax
from jax.experimental import pallas as pl
from jax.experimental.pallas import tpu as pltpu
from jax.experimental.pallas import tpu_sc as plsc

_N = 10000
_E = 320000
_D = 128
_H = 128
_C = 64

_NC = 2
_NS = 16
_NW = _NC * _NS
_EPW = _E // _NW
_B = 80
_NITER = _EPW // _B
_RPS = 632
_NPAD = _RPS * _NS


_G = 4
_NGRP = 31


def _sc_agg_body(x_hbm, src_hbm, dst_hbm, zeros_hbm, out_hbm,
                 sA0, sA1, sA2, sA3, dA0, dA1, dA2, dA3,
                 sB0, sB1, sB2, sB3, dB0, dB1, dB2, dB3,
                 rows0, rows1, rows2, rows3, agg_sh,
                 semIA, semIB, sg0, sg1, sg2, sg3, ss0, ss1, ss2, ss3):
    c = lax.axis_index("c")
    s = lax.axis_index("s")
    w = c * _NS + s
    srcA, dstA = (sA0, sA1, sA2, sA3), (dA0, dA1, dA2, dA3)
    srcB, dstB = (sB0, sB1, sB2, sB3), (dB0, dB1, dB2, dB3)
    rows = (rows0, rows1, rows2, rows3)
    sg = (sg0, sg1, sg2, sg3)
    ss = (ss0, ss1, ss2, ss3)

    pltpu.sync_copy(zeros_hbm.at[pl.ds(s * _RPS, _RPS)],
                    agg_sh.at[pl.ds(s * _RPS, _RPS)])
    plsc.subcore_barrier()

    def load_group(g, srcs, dsts, semi):
        base = w * _EPW + g * (_G * _B)
        for t in range(_G):
            pltpu.async_copy(src_hbm.at[pl.ds(base + t * _B, _B)], srcs[t],
                             semi)
            pltpu.async_copy(dst_hbm.at[pl.ds(base + t * _B, _B)], dsts[t],
                             semi)

    def wait_group_idx(srcs, dsts, semi):
        for t in range(_G):
            pltpu.make_async_copy(src_hbm.at[pl.ds(0, _B)], srcs[t],
                                  semi).wait()
            pltpu.make_async_copy(src_hbm.at[pl.ds(0, _B)], dsts[t],
                                  semi).wait()

    def process(srcs, dsts):
        gd = [pltpu.async_copy(x_hbm.at[srcs[t]], rows[t], sg[t])
              for t in range(_G)]
        sd = []
        for t in range(_G):
            gd[t].wait()
            sd.append(pltpu.async_copy(rows[t], agg_sh.at[dsts[t]], ss[t],
                                       add=True))
        for d in sd:
            d.wait()

    load_group(0, srcA, dstA, semIA)

    def body(k, carry):
        g0 = 2 * k
        load_group(g0 + 1, srcB, dstB, semIB)
        wait_group_idx(srcA, dstA, semIA)
        process(srcA, dstA)
        load_group(g0 + 2, srcA, dstA, semIA)
        wait_group_idx(srcB, dstB, semIB)
        process(srcB, dstB)
        return carry

    lax.fori_loop(0, (_NGRP - 1) // 2, body, 0)

    wait_group_idx(srcA, dstA, semIA)
    process(srcA, dstA)
    base = w * _EPW + (_NITER - 1) * _B
    pltpu.sync_copy(src_hbm.at[pl.ds(base, _B)], sA0)
    pltpu.sync_copy(dst_hbm.at[pl.ds(base, _B)], dA0)
    pltpu.async_copy(x_hbm.at[sA0], rows0, sg0).wait()
    pltpu.sync_copy(rows0, agg_sh.at[dA0], add=True)

    plsc.subcore_barrier()
    pltpu.sync_copy(agg_sh.at[pl.ds(s * _RPS, _RPS)],
                    out_hbm.at[c, pl.ds(s * _RPS, _RPS)])


@jax.jit
def _sc_segment_sum(x, src, dst, zeros):
    mesh = plsc.VectorSubcoreMesh(core_axis_name="c", subcore_axis_name="s")
    f = pl.kernel(
        _sc_agg_body,
        out_type=jax.ShapeDtypeStruct((_NC, _NPAD, _D), jnp.float32),
        mesh=mesh,
        scratch_types=[pltpu.VMEM((_B,), jnp.int32)] * 16
        + [pltpu.VMEM((_B, _D), jnp.float32)] * 4
        + [pltpu.VMEM_SHARED((_NPAD, _D), jnp.float32)]
        + [pltpu.SemaphoreType.DMA] * 10,
    )
    return f(x, src, dst, zeros)


def _mlp_body(eps_ref, x_ref, agg_ref, wa_ref, ba_ref, g_ref, be_ref,
              wb_ref, bb_ref, o_ref, *, last):
    agg = agg_ref[0, :_N, :] + agg_ref[1, :_N, :]
    h = (1.0 + eps_ref[0]) * x_ref[...] + agg
    t = jnp.dot(h, wa_ref[...], preferred_element_type=jnp.float32) + ba_ref[...]
    t = jnp.maximum(t, 0.0)
    mu = jnp.mean(t, axis=0, keepdims=True)
    var = jnp.mean((t - mu) ** 2, axis=0, keepdims=True)
    t = g_ref[...] * (t - mu) * lax.rsqrt(var + 1e-5) + be_ref[...]
    o = jnp.dot(t, wb_ref[...], preferred_element_type=jnp.float32) + bb_ref[...]
    if last:
        o = o - jnp.max(o, axis=-1, keepdims=True)
        o = o - jnp.log(jnp.sum(jnp.exp(o), axis=-1, keepdims=True))
    else:
        o = jnp.maximum(o, 0.0)
    o_ref[...] = o


def _mlp(eps, x, agg, wa, ba, g, be, wb, bb, *, last):
    cout = wb.shape[1]
    return pl.pallas_call(
        functools.partial(_mlp_body, last=last),
        out_shape=jax.ShapeDtypeStruct((_N, cout), jnp.float32),
        in_specs=[pl.BlockSpec(memory_space=pltpu.SMEM)]
        + [pl.BlockSpec(memory_space=pltpu.VMEM)] * 8,
        out_specs=pl.BlockSpec(memory_space=pltpu.VMEM),
    )(eps, x, agg, wa, ba, g, be, wb, bb)


def kernel(x, edge_index, eps1, W1a, b1a, g1, be1, W1b, b1b,
           eps2, W2a, b2a, g2, be2, W2b, b2b):
    ei = edge_index.astype(jnp.int32)
    zeros = jnp.zeros((_NPAD, _D), jnp.float32)
    e1 = jnp.reshape(eps1, (1,)).astype(jnp.float32)
    e2 = jnp.reshape(eps2, (1,)).astype(jnp.float32)

    src, dst = ei[0], ei[1]
    agg1 = _sc_segment_sum(x, src, dst, zeros)
    h1 = _mlp(e1, x, agg1, W1a, b1a.reshape(1, _H), g1.reshape(1, _H),
              be1.reshape(1, _H), W1b, b1b.reshape(1, _H), last=False)
    agg2 = _sc_segment_sum(h1, src, dst, zeros)
    out = _mlp(e2, h1, agg2, W2a, b2a.reshape(1, _H), g2.reshape(1, _H),
               be2.reshape(1, _H), W2b, b2b.reshape(1, _C), last=True)
    return out
